# Initial kernel scaffold; baseline (speedup 1.0000x reference)
import jax
import jax.numpy as jnp
from jax.experimental import pallas as pl


def kernel(x1, x2, edge1, edge2, y, params):
    raise NotImplementedError("write your pallas kernel here")


# speedup vs baseline: 1.4527x; 1.4527x over previous
import functools

import jax
import jax.numpy as jnp
from jax.experimental import pallas as pl
from jax.experimental.pallas import tpu as pltpu

N_NODES = 10000
K_TOP = 25
NEG = -3.0e38


def _round_up(x, m):
    return (x + m - 1) // m * m


def _mm_kernel(x_ref, w_ref, o_ref):
    o_ref[...] = jax.lax.dot_general(
        x_ref[...], w_ref[...], (((1,), (0,)), ((), ())),
        preferred_element_type=jnp.float32)


def _mm(x, w, bm=512):
    m, k = x.shape
    n = w.shape[1]
    mp = _round_up(m, bm)
    if mp != m:
        x = jnp.pad(x, ((0, mp - m), (0, 0)))
    out = pl.pallas_call(
        _mm_kernel,
        grid=(mp // bm,),
        in_specs=[
            pl.BlockSpec((bm, k), lambda i: (i, 0)),
            pl.BlockSpec((k, n), lambda i: (0, 0)),
        ],
        out_specs=pl.BlockSpec((bm, n), lambda i: (i, 0)),
        out_shape=jax.ShapeDtypeStruct((mp, n), jnp.float32),
    )(x, w)
    return out[:m]


def _combine_kernel(x_ref, w_ref, b_ref, a1_ref, d1_ref, a2_ref, d2_ref, o_ref):
    acc = jax.lax.dot_general(
        x_ref[...], w_ref[...], (((1,), (0,)), ((), ())),
        preferred_element_type=jnp.float32)
    acc = acc + b_ref[...] + a1_ref[...] * d1_ref[...] + a2_ref[...] * d2_ref[...]
    o_ref[...] = jnp.maximum(acc, 0.0)


def _combine(x, w, b, agg1, inv_d1, agg2, inv_d2, bm=512):
    m, k = x.shape
    n = w.shape[1]
    assert m % bm == 0
    return pl.pallas_call(
        _combine_kernel,
        grid=(m // bm,),
        in_specs=[
            pl.BlockSpec((bm, k), lambda i: (i, 0)),
            pl.BlockSpec((k, n), lambda i: (0, 0)),
            pl.BlockSpec((1, n), lambda i: (0, 0)),
            pl.BlockSpec((bm, n), lambda i: (i, 0)),
            pl.BlockSpec((bm, 1), lambda i: (i, 0)),
            pl.BlockSpec((bm, n), lambda i: (i, 0)),
            pl.BlockSpec((bm, 1), lambda i: (i, 0)),
        ],
        out_specs=pl.BlockSpec((bm, n), lambda i: (i, 0)),
        out_shape=jax.ShapeDtypeStruct((m, n), jnp.float32),
    )(x, w, b.reshape(1, n), agg1, inv_d1, agg2, inv_d2)


def _topk_kernel(hs_ref, ht_ref, o_ref, top_ref, *, bn, n_valid, k_pad):
    j = pl.program_id(1)
    nj = pl.num_programs(1)

    @pl.when(j == 0)
    def _init():
        top_ref[...] = jnp.full_like(top_ref, NEG)

    scores = jax.lax.dot_general(
        hs_ref[...], ht_ref[...], (((1,), (1,)), ((), ())),
        preferred_element_type=jnp.float32)

    col = jax.lax.broadcasted_iota(jnp.int32, scores.shape, 1) + j * bn
    scores = jnp.where(col < n_valid, scores, NEG)

    cand = jnp.concatenate([top_ref[...], scores], axis=1)
    w = cand.shape[1]
    lane = jax.lax.broadcasted_iota(jnp.int32, cand.shape, 1)
    for k in range(K_TOP):
        m = jnp.max(cand, axis=1, keepdims=True)
        top_ref[:, pl.ds(k, 1)] = m
        sel = jnp.where(cand == m, lane, w)
        first = jnp.min(sel, axis=1, keepdims=True)
        cand = jnp.where(lane == first, NEG, cand)

    @pl.when(j == nj - 1)
    def _softmax():
        t = top_ref[:, :K_TOP]
        mx = t[:, :1]
        e = jnp.exp(t - mx)
        s = jnp.sum(e, axis=1, keepdims=True)
        o_ref[...] = e / s


def _topk_softmax(h_s, h_t, bm=256, bn=512):
    m = h_s.shape[0]
    n = h_t.shape[0]
    mp = _round_up(m, bm)
    np_ = _round_up(n, bn)
    if mp != m:
        h_s = jnp.pad(h_s, ((0, mp - m), (0, 0)))
    if np_ != n:
        h_t = jnp.pad(h_t, ((0, np_ - n), (0, 0)))
    k_pad = 32
    out = pl.pallas_call(
        functools.partial(_topk_kernel, bn=bn, n_valid=n, k_pad=k_pad),
        grid=(mp // bm, np_ // bn),
        in_specs=[
            pl.BlockSpec((bm, h_s.shape[1]), lambda i, j: (i, 0)),
            pl.BlockSpec((bn, h_t.shape[1]), lambda i, j: (j, 0)),
        ],
        out_specs=pl.BlockSpec((bm, K_TOP), lambda i, j: (i, 0)),
        out_shape=jax.ShapeDtypeStruct((mp, K_TOP), jnp.float32),
        scratch_shapes=[pltpu.VMEM((bm, k_pad), jnp.float32)],
        compiler_params=pltpu.CompilerParams(
            dimension_semantics=("parallel", "arbitrary")),
    )(h_s, h_t)
    return out[:m]


def _gnn(x, src, dst, params, num_layers=3):
    n = x.shape[0]
    ones = jnp.ones(src.shape[0], dtype=jnp.float32)
    deg_in = jnp.clip(jax.ops.segment_sum(ones, dst, num_segments=n), 1.0)
    deg_out = jnp.clip(jax.ops.segment_sum(ones, src, num_segments=n), 1.0)
    inv_din = (1.0 / deg_in)[:, None]
    inv_dout = (1.0 / deg_out)[:, None]
    xs = [x]
    h = x
    for i in range(num_layers):
        w12 = jnp.concatenate([params[f"W1_{i}"], params[f"W2_{i}"]], axis=1)
        m12 = _mm(h, w12)
        m1, m2 = m12[:, :256], m12[:, 256:]
        agg1 = jax.ops.segment_sum(m1[src], dst, num_segments=n)
        agg2 = jax.ops.segment_sum(m2[dst], src, num_segments=n)
        h = _combine(h, params[f"Wr_{i}"], params[f"br_{i}"],
                     agg1, inv_din, agg2, inv_dout)
        xs.append(h)
    hcat = jnp.concatenate(xs, axis=1)
    return _mm(hcat, params["Wf"]) + params["bf"]


def kernel(x1, x2, edge1, edge2, y, params):
    e1 = jnp.concatenate([edge1[i] for i in range(edge1.shape[0])], axis=1)
    e2 = jnp.concatenate([edge2[i] for i in range(edge2.shape[0])], axis=1)
    n = x1.shape[0]
    npad = _round_up(n, 512)
    x1p = jnp.pad(x1, ((0, npad - n), (0, 0)))
    x2p = jnp.pad(x2, ((0, npad - n), (0, 0)))
    x = jnp.concatenate([x1p, x2p], axis=0)
    src = jnp.concatenate([e1[0], e2[0] + npad])
    dst = jnp.concatenate([e1[1], e2[1] + npad])
    h = _gnn(x, src, dst, params)
    h_s = h[:n]
    h_t = h[npad:npad + n]
    return _topk_softmax(h_s, h_t)


# speedup vs baseline: 2.9095x; 2.0027x over previous
import functools

import jax
import jax.numpy as jnp
from jax import lax
from jax.experimental import pallas as pl
from jax.experimental.pallas import tpu as pltpu
from jax.experimental.pallas import tpu_sc as plsc

N_NODES = 10000
NPAD = 10240
NDUMP = 8
E_EDGES = 160000
EPAD = 163840
CHUNK = 128
CHUNKS_PER_TILE = 80
K_TOP = 25
NEG = -3.0e38


def _round_up(x, m):
    return (x + m - 1) // m * m


def _mm_kernel(x_ref, w_ref, o_ref):
    o_ref[...] = jax.lax.dot_general(
        x_ref[...], w_ref[...], (((1,), (0,)), ((), ())),
        preferred_element_type=jnp.float32)


def _mm(x, w, bm=512):
    m, k = x.shape
    n = w.shape[1]
    mp = _round_up(m, bm)
    if mp != m:
        x = jnp.pad(x, ((0, mp - m), (0, 0)))
    out = pl.pallas_call(
        _mm_kernel,
        grid=(mp // bm,),
        in_specs=[
            pl.BlockSpec((bm, k), lambda i: (i, 0)),
            pl.BlockSpec((k, n), lambda i: (0, 0)),
        ],
        out_specs=pl.BlockSpec((bm, n), lambda i: (i, 0)),
        out_shape=jax.ShapeDtypeStruct((mp, n), jnp.float32),
    )(x, w)
    return out[:m]


def _mm4_kernel(x_ref, w_ref, o_ref):
    o_ref[0] = jax.lax.dot_general(
        x_ref[...], w_ref[...], (((1,), (0,)), ((), ())),
        preferred_element_type=jnp.float32)


def _mm4(x, w12, bm=512):
    m, k = x.shape
    assert m % bm == 0 and w12.shape == (k, 512)
    return pl.pallas_call(
        _mm4_kernel,
        grid=(m // bm, 4),
        in_specs=[
            pl.BlockSpec((bm, k), lambda i, h: (i, 0)),
            pl.BlockSpec((k, 128), lambda i, h: (0, h)),
        ],
        out_specs=pl.BlockSpec((1, bm, 128), lambda i, h: (h, i, 0)),
        out_shape=jax.ShapeDtypeStruct((4, m, 128), jnp.float32),
        compiler_params=pltpu.CompilerParams(
            dimension_semantics=("parallel", "parallel")),
    )(x, w12)


def _combine_kernel(x_ref, w_ref, b_ref, a1_ref, a2_ref, d_ref, o_ref):
    acc = jax.lax.dot_general(
        x_ref[...], w_ref[...], (((1,), (0,)), ((), ())),
        preferred_element_type=jnp.float32)
    a1 = jnp.concatenate([a1_ref[0], a1_ref[1]], axis=1)
    a2 = jnp.concatenate([a2_ref[0], a2_ref[1]], axis=1)
    din = jnp.maximum(d_ref[0, 0][:, :1], 1.0)
    dout = jnp.maximum(d_ref[1, 0][:, :1], 1.0)
    acc = acc + b_ref[...] + a1 / din + a2 / dout
    o_ref[...] = jnp.maximum(acc, 0.0)


def _combine(x, w, b, a1, a2, deg, bm=512):
    m, k = x.shape
    n = w.shape[1]
    nb = NPAD // bm
    return pl.pallas_call(
        _combine_kernel,
        grid=(m // bm,),
        in_specs=[
            pl.BlockSpec((bm, k), lambda i: (i, 0)),
            pl.BlockSpec((k, n), lambda i: (0, 0)),
            pl.BlockSpec((1, n), lambda i: (0, 0)),
            pl.BlockSpec((2, bm, 128), lambda i: (0, i, 0)),
            pl.BlockSpec((2, bm, 128), lambda i: (0, i, 0)),
            pl.BlockSpec((2, 1, bm, 128), lambda i: (0, i // nb, i % nb, 0)),
        ],
        out_specs=pl.BlockSpec((bm, n), lambda i: (i, 0)),
        out_shape=jax.ShapeDtypeStruct((m, n), jnp.float32),
    )(x, w, b.reshape(1, n), a1, a2, deg)


_MESH = plsc.VectorSubcoreMesh(core_axis_name="c", subcore_axis_name="s")


def _sc_scatter_body(mout, gidx, sidx, zrows, out1, out2, gv, sv, rows, acc):
    c = lax.axis_index("c")
    s = lax.axis_index("s")
    r0 = s * (NPAD // 16)
    nr = NPAD // 16
    pltpu.sync_copy(zrows.at[pl.ds(r0, nr)], acc.at[pl.ds(r0, nr)])

    @pl.when(s == 0)
    def _zero_dump():
        pltpu.sync_copy(zrows.at[pl.ds(NPAD, NDUMP)], acc.at[pl.ds(NPAD, NDUMP)])

    plsc.subcore_barrier()

    for p in range(4):
        tbl = mout.at[2 * (p % 2) + c]
        out = out1 if p % 2 == 0 else out2
        row_off = 0 if p < 2 else NPAD
        pltpu.sync_copy(gidx.at[p, pl.ds(s * CHUNKS_PER_TILE, CHUNKS_PER_TILE)], gv)
        pltpu.sync_copy(sidx.at[p, pl.ds(s * CHUNKS_PER_TILE, CHUNKS_PER_TILE)], sv)

        def chunk(j, carry):
            pltpu.sync_copy(tbl.at[gv.at[j]], rows)
            pltpu.sync_copy(rows, acc.at[sv.at[j]], add=True)
            return carry

        lax.fori_loop(0, CHUNKS_PER_TILE, chunk, 0)
        plsc.subcore_barrier()
        pltpu.sync_copy(acc.at[pl.ds(r0, nr)], out.at[c, pl.ds(row_off + r0, nr)])
        if p < 3:
            pltpu.sync_copy(zrows.at[pl.ds(r0, nr)], acc.at[pl.ds(r0, nr)])
        plsc.subcore_barrier()


def _sc_scatter(mout, gidx, sidx, zrows):
    f = pl.kernel(
        _sc_scatter_body,
        out_type=(
            jax.ShapeDtypeStruct((2, 2 * NPAD, 128), jnp.float32),
            jax.ShapeDtypeStruct((2, 2 * NPAD, 128), jnp.float32),
        ),
        mesh=_MESH,
        scratch_types=[
            pltpu.VMEM((CHUNKS_PER_TILE, CHUNK), jnp.int32),
            pltpu.VMEM((CHUNKS_PER_TILE, CHUNK), jnp.int32),
            pltpu.VMEM((CHUNK, 128), jnp.float32),
            pltpu.VMEM_SHARED((NPAD + NDUMP, 128), jnp.float32),
        ],
    )
    return f(mout, gidx, sidx, zrows)


def _sc_degrees_body(sidx, ones_h, zrows, outdeg, sv, ones_v, accd):
    c = lax.axis_index("c")
    s = lax.axis_index("s")
    r0 = s * (NPAD // 16)
    nr = NPAD // 16
    pltpu.sync_copy(ones_h, ones_v)
    pltpu.sync_copy(zrows.at[pl.ds(r0, nr)], accd.at[pl.ds(r0, nr)])

    @pl.when(s == 0)
    def _zero_dump():
        pltpu.sync_copy(zrows.at[pl.ds(NPAD, NDUMP)], accd.at[pl.ds(NPAD, NDUMP)])

    plsc.subcore_barrier()
    for d in range(2):
        pltpu.sync_copy(sidx.at[2 * c + d, pl.ds(s * CHUNKS_PER_TILE, CHUNKS_PER_TILE)], sv)

        def chunk(j, carry):
            pltpu.sync_copy(ones_v, accd.at[sv.at[j]], add=True)
            return carry

        lax.fori_loop(0, CHUNKS_PER_TILE, chunk, 0)
        plsc.subcore_barrier()
        pltpu.sync_copy(accd.at[pl.ds(r0, nr)], outdeg.at[d, c, pl.ds(r0, nr)])
        if d == 0:
            pltpu.sync_copy(zrows.at[pl.ds(r0, nr)], accd.at[pl.ds(r0, nr)])
        plsc.subcore_barrier()


def _sc_degrees(sidx, ones_h, zrows):
    f = pl.kernel(
        _sc_degrees_body,
        out_type=jax.ShapeDtypeStruct((2, 2, NPAD, 128), jnp.float32),
        mesh=_MESH,
        scratch_types=[
            pltpu.VMEM((CHUNKS_PER_TILE, CHUNK), jnp.int32),
            pltpu.VMEM((CHUNK, 128), jnp.float32),
            pltpu.VMEM_SHARED((NPAD + NDUMP, 128), jnp.float32),
        ],
    )
    return f(sidx, ones_h, zrows)


def _topk_kernel(hs_ref, ht_ref, o_ref, top_ref, *, bn, n_valid):
    j = pl.program_id(1)
    nj = pl.num_programs(1)

    @pl.when(j == 0)
    def _init():
        top_ref[...] = jnp.full_like(top_ref, NEG)

    scores = jax.lax.dot_general(
        hs_ref[...], ht_ref[...], (((1,), (1,)), ((), ())),
        preferred_element_type=jnp.float32)

    col = jax.lax.broadcasted_iota(jnp.int32, scores.shape, 1) + j * bn
    scores = jnp.where(col < n_valid, scores, NEG)

    cand = jnp.concatenate([top_ref[...], scores], axis=1)
    w = cand.shape[1]
    lane = jax.lax.broadcasted_iota(jnp.int32, cand.shape, 1)
    for k in range(K_TOP):
        m = jnp.max(cand, axis=1, keepdims=True)
        top_ref[:, pl.ds(k, 1)] = m
        sel = jnp.where(cand == m, lane, w)
        first = jnp.min(sel, axis=1, keepdims=True)
        cand = jnp.where(lane == first, NEG, cand)

    @pl.when(j == nj - 1)
    def _softmax():
        t = top_ref[:, :K_TOP]
        mx = t[:, :1]
        e = jnp.exp(t - mx)
        s = jnp.sum(e, axis=1, keepdims=True)
        o_ref[...] = e / s


def _topk_softmax(h_s, h_t, bm=256, bn=512):
    m = h_s.shape[0]
    n = h_t.shape[0]
    mp = _round_up(m, bm)
    np_ = _round_up(n, bn)
    if mp != m:
        h_s = jnp.pad(h_s, ((0, mp - m), (0, 0)))
    if np_ != n:
        h_t = jnp.pad(h_t, ((0, np_ - n), (0, 0)))
    out = pl.pallas_call(
        functools.partial(_topk_kernel, bn=bn, n_valid=n),
        grid=(mp // bm, np_ // bn),
        in_specs=[
            pl.BlockSpec((bm, h_s.shape[1]), lambda i, j: (i, 0)),
            pl.BlockSpec((bn, h_t.shape[1]), lambda i, j: (j, 0)),
        ],
        out_specs=pl.BlockSpec((bm, K_TOP), lambda i, j: (i, 0)),
        out_shape=jax.ShapeDtypeStruct((mp, K_TOP), jnp.float32),
        scratch_shapes=[pltpu.VMEM((bm, 32), jnp.float32)],
        compiler_params=pltpu.CompilerParams(
            dimension_semantics=("parallel", "arbitrary")),
    )(h_s, h_t)
    return out[:m]


def _pad_idx(idx, fill):
    return jnp.concatenate([idx, fill.astype(jnp.int32)])


def kernel(x1, x2, edge1, edge2, y, params):
    e1 = jnp.concatenate([edge1[i] for i in range(edge1.shape[0])], axis=1)
    e2 = jnp.concatenate([edge2[i] for i in range(edge2.shape[0])], axis=1)
    n = x1.shape[0]
    x1p = jnp.pad(x1, ((0, NPAD - n), (0, 0)))
    x2p = jnp.pad(x2, ((0, NPAD - n), (0, 0)))
    x = jnp.concatenate([x1p, x2p], axis=0)

    npad_e = EPAD - E_EDGES
    ar = jnp.arange(npad_e, dtype=jnp.int32)
    gfill = ar % 64
    sfill = NPAD + (ar % NDUMP)
    src1, dst1 = e1[0], e1[1]
    src2, dst2 = e2[0], e2[1]
    gidx = jnp.stack([
        _pad_idx(src1, gfill),
        _pad_idx(dst1, gfill),
        _pad_idx(src2 + NPAD, gfill),
        _pad_idx(dst2 + NPAD, gfill),
    ]).reshape(4, EPAD // CHUNK, CHUNK)
    sidx = jnp.stack([
        _pad_idx(dst1, sfill),
        _pad_idx(src1, sfill),
        _pad_idx(dst2, sfill),
        _pad_idx(src2, sfill),
    ]).reshape(4, EPAD // CHUNK, CHUNK)

    zrows = jnp.zeros((NPAD + NDUMP, 128), jnp.float32)
    ones_h = jnp.ones((CHUNK, 128), jnp.float32)
    deg = _sc_degrees(sidx, ones_h, zrows)

    h = x
    xs = [x]
    for i in range(3):
        w12 = jnp.concatenate([params[f"W1_{i}"], params[f"W2_{i}"]], axis=1)
        mout = _mm4(h, w12)
        a1, a2 = _sc_scatter(mout, gidx, sidx, zrows)
        h = _combine(h, params[f"Wr_{i}"], params[f"br_{i}"], a1, a2, deg)
        xs.append(h)
    hcat = jnp.concatenate(xs, axis=1)
    hf = _mm(hcat, params["Wf"]) + params["bf"]
    h_s = hf[:n]
    h_t = hf[NPAD:NPAD + n]
    return _topk_softmax(h_s, h_t)


# speedup vs baseline: 5.5417x; 1.9047x over previous
import functools

import jax
import jax.numpy as jnp
from jax import lax
from jax.experimental import pallas as pl
from jax.experimental.pallas import tpu as pltpu
from jax.experimental.pallas import tpu_sc as plsc

N_NODES = 10000
NPAD = 10240
NDUMP = 8
E_EDGES = 160000
EPAD = 163840
CHUNK = 128
CHUNKS_PER_TILE = 80
K_TOP = 25
NEG = -3.0e38


def _round_up(x, m):
    return (x + m - 1) // m * m


def _mm_kernel(x_ref, w_ref, o_ref):
    o_ref[...] = jax.lax.dot_general(
        x_ref[...], w_ref[...], (((1,), (0,)), ((), ())),
        preferred_element_type=jnp.float32)


def _mm(x, w, bm=512):
    m, k = x.shape
    n = w.shape[1]
    mp = _round_up(m, bm)
    if mp != m:
        x = jnp.pad(x, ((0, mp - m), (0, 0)))
    out = pl.pallas_call(
        _mm_kernel,
        grid=(mp // bm,),
        in_specs=[
            pl.BlockSpec((bm, k), lambda i: (i, 0)),
            pl.BlockSpec((k, n), lambda i: (0, 0)),
        ],
        out_specs=pl.BlockSpec((bm, n), lambda i: (i, 0)),
        out_shape=jax.ShapeDtypeStruct((mp, n), jnp.float32),
    )(x, w)
    return out[:m]


def _mm4_kernel(x_ref, w_ref, o_ref):
    o_ref[0] = jax.lax.dot_general(
        x_ref[...], w_ref[...], (((1,), (0,)), ((), ())),
        preferred_element_type=jnp.float32)


def _mm4(x, w12, bm=512):
    m, k = x.shape
    assert m % bm == 0 and w12.shape == (k, 512)
    return pl.pallas_call(
        _mm4_kernel,
        grid=(m // bm, 4),
        in_specs=[
            pl.BlockSpec((bm, k), lambda i, h: (i, 0)),
            pl.BlockSpec((k, 128), lambda i, h: (0, h)),
        ],
        out_specs=pl.BlockSpec((1, bm, 128), lambda i, h: (h, i, 0)),
        out_shape=jax.ShapeDtypeStruct((4, m, 128), jnp.float32),
        compiler_params=pltpu.CompilerParams(
            dimension_semantics=("parallel", "parallel")),
    )(x, w12)


def _combine_kernel(x_ref, w_ref, b_ref, a1_ref, a2_ref, d_ref, o_ref):
    acc = jax.lax.dot_general(
        x_ref[...], w_ref[...], (((1,), (0,)), ((), ())),
        preferred_element_type=jnp.float32)
    a1 = jnp.concatenate([a1_ref[0], a1_ref[1]], axis=1)
    a2 = jnp.concatenate([a2_ref[0], a2_ref[1]], axis=1)
    din = jnp.maximum(d_ref[0, 0][:, :1], 1.0)
    dout = jnp.maximum(d_ref[1, 0][:, :1], 1.0)
    acc = acc + b_ref[...] + a1 / din + a2 / dout
    o_ref[...] = jnp.maximum(acc, 0.0)


def _combine(x, w, b, a1, a2, deg, bm=512):
    m, k = x.shape
    n = w.shape[1]
    nb = NPAD // bm
    return pl.pallas_call(
        _combine_kernel,
        grid=(m // bm,),
        in_specs=[
            pl.BlockSpec((bm, k), lambda i: (i, 0)),
            pl.BlockSpec((k, n), lambda i: (0, 0)),
            pl.BlockSpec((1, n), lambda i: (0, 0)),
            pl.BlockSpec((2, bm, 128), lambda i: (0, i, 0)),
            pl.BlockSpec((2, bm, 128), lambda i: (0, i, 0)),
            pl.BlockSpec((2, 1, bm, 128), lambda i: (0, i // nb, i % nb, 0)),
        ],
        out_specs=pl.BlockSpec((bm, n), lambda i: (i, 0)),
        out_shape=jax.ShapeDtypeStruct((m, n), jnp.float32),
    )(x, w, b.reshape(1, n), a1, a2, deg)


@functools.cache
def _sc_mesh():
    return plsc.VectorSubcoreMesh(core_axis_name="c", subcore_axis_name="s")


def _sc_scatter_body(mout, gidx, sidx, zrows, out1, out2, gv, sv, rows, acc):
    c = lax.axis_index("c")
    s = lax.axis_index("s")
    r0 = s * (NPAD // 16)
    nr = NPAD // 16
    pltpu.sync_copy(zrows.at[pl.ds(r0, nr)], acc.at[pl.ds(r0, nr)])

    @pl.when(s == 0)
    def _zero_dump():
        pltpu.sync_copy(zrows.at[pl.ds(NPAD, NDUMP)], acc.at[pl.ds(NPAD, NDUMP)])

    plsc.subcore_barrier()

    for p in range(4):
        tbl = mout.at[2 * (p % 2) + c]
        out = out1 if p % 2 == 0 else out2
        row_off = 0 if p < 2 else NPAD
        pltpu.sync_copy(gidx.at[p, pl.ds(s * CHUNKS_PER_TILE, CHUNKS_PER_TILE)], gv)
        pltpu.sync_copy(sidx.at[p, pl.ds(s * CHUNKS_PER_TILE, CHUNKS_PER_TILE)], sv)

        def chunk(j, carry):
            pltpu.sync_copy(tbl.at[gv.at[j]], rows)
            pltpu.sync_copy(rows, acc.at[sv.at[j]], add=True)
            return carry

        lax.fori_loop(0, CHUNKS_PER_TILE, chunk, 0)
        plsc.subcore_barrier()
        pltpu.sync_copy(acc.at[pl.ds(r0, nr)], out.at[c, pl.ds(row_off + r0, nr)])
        if p < 3:
            pltpu.sync_copy(zrows.at[pl.ds(r0, nr)], acc.at[pl.ds(r0, nr)])
        plsc.subcore_barrier()


def _sc_scatter(mout, gidx, sidx, zrows):
    f = pl.kernel(
        _sc_scatter_body,
        out_type=(
            jax.ShapeDtypeStruct((2, 2 * NPAD, 128), jnp.float32),
            jax.ShapeDtypeStruct((2, 2 * NPAD, 128), jnp.float32),
        ),
        mesh=_sc_mesh(),
        scratch_types=[
            pltpu.VMEM((CHUNKS_PER_TILE, CHUNK), jnp.int32),
            pltpu.VMEM((CHUNKS_PER_TILE, CHUNK), jnp.int32),
            pltpu.VMEM((CHUNK, 128), jnp.float32),
            pltpu.VMEM_SHARED((NPAD + NDUMP, 128), jnp.float32),
        ],
    )
    return f(mout, gidx, sidx, zrows)


def _sc_degrees_body(sidx, ones_h, zrows, outdeg, sv, ones_v, accd):
    c = lax.axis_index("c")
    s = lax.axis_index("s")
    r0 = s * (NPAD // 16)
    nr = NPAD // 16
    pltpu.sync_copy(ones_h, ones_v)
    pltpu.sync_copy(zrows.at[pl.ds(r0, nr)], accd.at[pl.ds(r0, nr)])

    @pl.when(s == 0)
    def _zero_dump():
        pltpu.sync_copy(zrows.at[pl.ds(NPAD, NDUMP)], accd.at[pl.ds(NPAD, NDUMP)])

    plsc.subcore_barrier()
    for d in range(2):
        pltpu.sync_copy(sidx.at[2 * c + d, pl.ds(s * CHUNKS_PER_TILE, CHUNKS_PER_TILE)], sv)

        def chunk(j, carry):
            pltpu.sync_copy(ones_v, accd.at[sv.at[j]], add=True)
            return carry

        lax.fori_loop(0, CHUNKS_PER_TILE, chunk, 0)
        plsc.subcore_barrier()
        pltpu.sync_copy(accd.at[pl.ds(r0, nr)], outdeg.at[d, c, pl.ds(r0, nr)])
        if d == 0:
            pltpu.sync_copy(zrows.at[pl.ds(r0, nr)], accd.at[pl.ds(r0, nr)])
        plsc.subcore_barrier()


def _sc_degrees(sidx, ones_h, zrows):
    f = pl.kernel(
        _sc_degrees_body,
        out_type=jax.ShapeDtypeStruct((2, 2, NPAD, 128), jnp.float32),
        mesh=_sc_mesh(),
        scratch_types=[
            pltpu.VMEM((CHUNKS_PER_TILE, CHUNK), jnp.int32),
            pltpu.VMEM((CHUNK, 128), jnp.float32),
            pltpu.VMEM_SHARED((NPAD + NDUMP, 128), jnp.float32),
        ],
    )
    return f(sidx, ones_h, zrows)


def _oddeven_comparators(n):
    res = []

    def merge(lo, m, r):
        step = r * 2
        if step < m:
            merge(lo, m, step)
            merge(lo + r, m, step)
            for i in range(lo + r, lo + m - r, step):
                res.append((i, i + r))
        else:
            res.append((lo, lo + r))

    def sort(lo, m):
        if m > 1:
            h = m // 2
            sort(lo, h)
            sort(lo + h, h)
            merge(lo, m, 1)

    sort(0, n)
    return res


_NSEG = 16


def _topk_kernel(hs_ref, ht_ref, o_ref, strip_ref, *, bn, n_valid):
    j = pl.program_id(1)
    nj = pl.num_programs(1)

    scores = jax.lax.dot_general(
        hs_ref[...], ht_ref[...], (((1,), (1,)), ((), ())),
        preferred_element_type=jnp.float32)

    col = jax.lax.broadcasted_iota(jnp.int32, scores.shape, 1) + j * bn
    scores = jnp.where(col < n_valid, scores, NEG)
    strip_ref[j] = scores

    @pl.when(j == nj - 1)
    def _select():
        a = [strip_ref[t] for t in range(_NSEG)]
        for (p, q) in _oddeven_comparators(_NSEG):
            hi = jnp.maximum(a[p], a[q])
            lo = jnp.minimum(a[p], a[q])
            a[p], a[q] = hi, lo
        w = a[0].shape[1]
        lane = jax.lax.broadcasted_iota(jnp.int32, a[0].shape, 1)
        vals = []
        for _ in range(K_TOP):
            m = jnp.max(a[0], axis=1, keepdims=True)
            vals.append(m)
            sel = jnp.where(a[0] == m, lane, w)
            first = jnp.min(sel, axis=1, keepdims=True)
            hit = lane == first
            for t in range(_NSEG - 1):
                a[t] = jnp.where(hit, a[t + 1], a[t])
            a[_NSEG - 1] = jnp.where(hit, NEG, a[_NSEG - 1])
        t = jnp.concatenate(vals, axis=1)
        e = jnp.exp(t - t[:, :1])
        s = jnp.sum(e, axis=1, keepdims=True)
        o_ref[...] = e / s


def _topk_softmax(h_s, h_t, bm=256, bn=640):
    m = h_s.shape[0]
    n = h_t.shape[0]
    mp = _round_up(m, bm)
    np_ = _round_up(n, bn * _NSEG) if n % (bn * _NSEG) else n
    if mp != m:
        h_s = jnp.pad(h_s, ((0, mp - m), (0, 0)))
    if np_ != n:
        h_t = jnp.pad(h_t, ((0, np_ - n), (0, 0)))
    assert np_ == bn * _NSEG
    out = pl.pallas_call(
        functools.partial(_topk_kernel, bn=bn, n_valid=n),
        grid=(mp // bm, _NSEG),
        in_specs=[
            pl.BlockSpec((bm, h_s.shape[1]), lambda i, j: (i, 0)),
            pl.BlockSpec((bn, h_t.shape[1]), lambda i, j: (j, 0)),
        ],
        out_specs=pl.BlockSpec((bm, K_TOP), lambda i, j: (i, 0)),
        out_shape=jax.ShapeDtypeStruct((mp, K_TOP), jnp.float32),
        scratch_shapes=[pltpu.VMEM((_NSEG, bm, bn), jnp.float32)],
        compiler_params=pltpu.CompilerParams(
            dimension_semantics=("parallel", "arbitrary")),
    )(h_s, h_t)
    return out[:m]


def _pad_idx(idx, fill):
    return jnp.concatenate([idx, fill.astype(jnp.int32)])


def kernel(x1, x2, edge1, edge2, y, params):
    e1 = jnp.concatenate([edge1[i] for i in range(edge1.shape[0])], axis=1)
    e2 = jnp.concatenate([edge2[i] for i in range(edge2.shape[0])], axis=1)
    n = x1.shape[0]
    x1p = jnp.pad(x1, ((0, NPAD - n), (0, 0)))
    x2p = jnp.pad(x2, ((0, NPAD - n), (0, 0)))
    x = jnp.concatenate([x1p, x2p], axis=0)

    npad_e = EPAD - E_EDGES
    ar = jnp.arange(npad_e, dtype=jnp.int32)
    gfill = ar % 64
    sfill = NPAD + (ar % NDUMP)
    src1, dst1 = e1[0], e1[1]
    src2, dst2 = e2[0], e2[1]
    gidx = jnp.stack([
        _pad_idx(src1, gfill),
        _pad_idx(dst1, gfill),
        _pad_idx(src2 + NPAD, gfill),
        _pad_idx(dst2 + NPAD, gfill),
    ]).reshape(4, EPAD // CHUNK, CHUNK)
    sidx = jnp.stack([
        _pad_idx(dst1, sfill),
        _pad_idx(src1, sfill),
        _pad_idx(dst2, sfill),
        _pad_idx(src2, sfill),
    ]).reshape(4, EPAD // CHUNK, CHUNK)

    zrows = jnp.zeros((NPAD + NDUMP, 128), jnp.float32)
    ones_h = jnp.ones((CHUNK, 128), jnp.float32)
    deg = _sc_degrees(sidx, ones_h, zrows)

    h = x
    xs = [x]
    for i in range(3):
        w12 = jnp.concatenate([params[f"W1_{i}"], params[f"W2_{i}"]], axis=1)
        mout = _mm4(h, w12)
        a1, a2 = _sc_scatter(mout, gidx, sidx, zrows)
        h = _combine(h, params[f"Wr_{i}"], params[f"br_{i}"], a1, a2, deg)
        xs.append(h)
    hcat = jnp.concatenate(xs, axis=1)
    hf = _mm(hcat, params["Wf"]) + params["bf"]
    h_s = hf[:n]
    h_t = hf[NPAD:NPAD + n]
    return _topk_softmax(h_s, h_t)


# speedup vs baseline: 6.3467x; 1.1453x over previous
import functools

import jax
import jax.numpy as jnp
from jax import lax
from jax.experimental import pallas as pl
from jax.experimental.pallas import tpu as pltpu
from jax.experimental.pallas import tpu_sc as plsc

N_NODES = 10000
NPAD = 10240
NDUMP = 8
E_EDGES = 160000
EPAD = 163840
CHUNK = 128
CHUNKS_PER_TILE = 80
K_TOP = 25
NEG = -3.0e38


def _round_up(x, m):
    return (x + m - 1) // m * m


def _mm_kernel(x_ref, w_ref, o_ref):
    o_ref[...] = jax.lax.dot_general(
        x_ref[...], w_ref[...], (((1,), (0,)), ((), ())),
        preferred_element_type=jnp.float32)


def _mm(x, w, bm=512):
    m, k = x.shape
    n = w.shape[1]
    mp = _round_up(m, bm)
    if mp != m:
        x = jnp.pad(x, ((0, mp - m), (0, 0)))
    out = pl.pallas_call(
        _mm_kernel,
        grid=(mp // bm,),
        in_specs=[
            pl.BlockSpec((bm, k), lambda i: (i, 0)),
            pl.BlockSpec((k, n), lambda i: (0, 0)),
        ],
        out_specs=pl.BlockSpec((bm, n), lambda i: (i, 0)),
        out_shape=jax.ShapeDtypeStruct((mp, n), jnp.float32),
    )(x, w)
    return out[:m]


def _mm4_kernel(x_ref, w_ref, o_ref):
    o_ref[0] = jax.lax.dot_general(
        x_ref[...], w_ref[...], (((1,), (0,)), ((), ())),
        preferred_element_type=jnp.float32)


def _mm4(x, w12, bm=512):
    m, k = x.shape
    assert m % bm == 0 and w12.shape == (k, 512)
    return pl.pallas_call(
        _mm4_kernel,
        grid=(m // bm, 4),
        in_specs=[
            pl.BlockSpec((bm, k), lambda i, h: (i, 0)),
            pl.BlockSpec((k, 128), lambda i, h: (0, h)),
        ],
        out_specs=pl.BlockSpec((1, bm, 128), lambda i, h: (h, i, 0)),
        out_shape=jax.ShapeDtypeStruct((4, m, 128), jnp.float32),
        compiler_params=pltpu.CompilerParams(
            dimension_semantics=("parallel", "parallel")),
    )(x, w12)


def _combine_kernel(x_ref, w_ref, b_ref, a1_ref, a2_ref, d_ref, o_ref):
    acc = jax.lax.dot_general(
        x_ref[...], w_ref[...], (((1,), (0,)), ((), ())),
        preferred_element_type=jnp.float32)
    a1 = jnp.concatenate([a1_ref[0], a1_ref[1]], axis=1)
    a2 = jnp.concatenate([a2_ref[0], a2_ref[1]], axis=1)
    din = jnp.maximum(d_ref[0, 0][:, :1], 1.0)
    dout = jnp.maximum(d_ref[1, 0][:, :1], 1.0)
    acc = acc + b_ref[...] + a1 / din + a2 / dout
    o_ref[...] = jnp.maximum(acc, 0.0)


def _combine(x, w, b, a1, a2, deg, bm=512):
    m, k = x.shape
    n = w.shape[1]
    nb = NPAD // bm
    return pl.pallas_call(
        _combine_kernel,
        grid=(m // bm,),
        in_specs=[
            pl.BlockSpec((bm, k), lambda i: (i, 0)),
            pl.BlockSpec((k, n), lambda i: (0, 0)),
            pl.BlockSpec((1, n), lambda i: (0, 0)),
            pl.BlockSpec((2, bm, 128), lambda i: (0, i, 0)),
            pl.BlockSpec((2, bm, 128), lambda i: (0, i, 0)),
            pl.BlockSpec((2, 1, bm, 128), lambda i: (0, i // nb, i % nb, 0)),
        ],
        out_specs=pl.BlockSpec((bm, n), lambda i: (i, 0)),
        out_shape=jax.ShapeDtypeStruct((m, n), jnp.float32),
    )(x, w, b.reshape(1, n), a1, a2, deg)


@functools.cache
def _sc_mesh():
    return plsc.VectorSubcoreMesh(core_axis_name="c", subcore_axis_name="s")


def _sc_scatter_body(mout, gidx, sidx, zrows, out1, out2, gv, sv, rows0, rows1,
                     sem0, sem1, acc):
    c = lax.axis_index("c")
    s = lax.axis_index("s")
    r0 = s * (NPAD // 16)
    nr = NPAD // 16
    pltpu.sync_copy(zrows.at[pl.ds(r0, nr)], acc.at[pl.ds(r0, nr)])

    @pl.when(s == 0)
    def _zero_dump():
        pltpu.sync_copy(zrows.at[pl.ds(NPAD, NDUMP)], acc.at[pl.ds(NPAD, NDUMP)])

    plsc.subcore_barrier()

    for p in range(4):
        tbl = mout.at[2 * (p % 2) + c]
        out = out1 if p % 2 == 0 else out2
        row_off = 0 if p < 2 else NPAD
        for h in range(2):
            base = s * CHUNKS_PER_TILE + h * (CHUNKS_PER_TILE // 2)
            nh = CHUNKS_PER_TILE // 2
            pltpu.sync_copy(gidx.at[p, pl.ds(base, nh)], gv)
            pltpu.sync_copy(sidx.at[p, pl.ds(base, nh)], sv)
            pltpu.async_copy(tbl.at[gv.at[0]], rows0, sem0)

            def chunk(i, carry):
                j0 = 2 * i
                j1 = 2 * i + 1
                pltpu.make_async_copy(tbl.at[gv.at[j0]], rows0, sem0).wait()
                pltpu.async_copy(tbl.at[gv.at[j1]], rows1, sem1)
                pltpu.sync_copy(rows0, acc.at[sv.at[j0]], add=True)
                pltpu.make_async_copy(tbl.at[gv.at[j1]], rows1, sem1).wait()

                @pl.when(j0 + 2 < nh)
                def _next():
                    pltpu.async_copy(tbl.at[gv.at[j0 + 2]], rows0, sem0)

                pltpu.sync_copy(rows1, acc.at[sv.at[j1]], add=True)
                return carry

            lax.fori_loop(0, nh // 2, chunk, 0)
        plsc.subcore_barrier()
        pltpu.sync_copy(acc.at[pl.ds(r0, nr)], out.at[c, pl.ds(row_off + r0, nr)])
        if p < 3:
            pltpu.sync_copy(zrows.at[pl.ds(r0, nr)], acc.at[pl.ds(r0, nr)])
        plsc.subcore_barrier()


def _sc_scatter(mout, gidx, sidx, zrows):
    f = pl.kernel(
        _sc_scatter_body,
        out_type=(
            jax.ShapeDtypeStruct((2, 2 * NPAD, 128), jnp.float32),
            jax.ShapeDtypeStruct((2, 2 * NPAD, 128), jnp.float32),
        ),
        mesh=_sc_mesh(),
        scratch_types=[
            pltpu.VMEM((CHUNKS_PER_TILE // 2, CHUNK), jnp.int32),
            pltpu.VMEM((CHUNKS_PER_TILE // 2, CHUNK), jnp.int32),
            pltpu.VMEM((CHUNK, 128), jnp.float32),
            pltpu.VMEM((CHUNK, 128), jnp.float32),
            pltpu.SemaphoreType.DMA,
            pltpu.SemaphoreType.DMA,
            pltpu.VMEM_SHARED((NPAD + NDUMP, 128), jnp.float32),
        ],
    )
    return f(mout, gidx, sidx, zrows)


def _sc_degrees_body(sidx, ones_h, zrows, outdeg, sv, ones_v, accd):
    c = lax.axis_index("c")
    s = lax.axis_index("s")
    r0 = s * (NPAD // 16)
    nr = NPAD // 16
    pltpu.sync_copy(ones_h, ones_v)
    pltpu.sync_copy(zrows.at[pl.ds(r0, nr)], accd.at[pl.ds(r0, nr)])

    @pl.when(s == 0)
    def _zero_dump():
        pltpu.sync_copy(zrows.at[pl.ds(NPAD, NDUMP)], accd.at[pl.ds(NPAD, NDUMP)])

    plsc.subcore_barrier()
    for d in range(2):
        pltpu.sync_copy(sidx.at[2 * c + d, pl.ds(s * CHUNKS_PER_TILE, CHUNKS_PER_TILE)], sv)

        def chunk(j, carry):
            pltpu.sync_copy(ones_v, accd.at[sv.at[j]], add=True)
            return carry

        lax.fori_loop(0, CHUNKS_PER_TILE, chunk, 0)
        plsc.subcore_barrier()
        pltpu.sync_copy(accd.at[pl.ds(r0, nr)], outdeg.at[d, c, pl.ds(r0, nr)])
        if d == 0:
            pltpu.sync_copy(zrows.at[pl.ds(r0, nr)], accd.at[pl.ds(r0, nr)])
        plsc.subcore_barrier()


def _sc_degrees(sidx, ones_h, zrows):
    f = pl.kernel(
        _sc_degrees_body,
        out_type=jax.ShapeDtypeStruct((2, 2, NPAD, 128), jnp.float32),
        mesh=_sc_mesh(),
        scratch_types=[
            pltpu.VMEM((CHUNKS_PER_TILE, CHUNK), jnp.int32),
            pltpu.VMEM((CHUNK, 128), jnp.float32),
            pltpu.VMEM_SHARED((NPAD + NDUMP, 128), jnp.float32),
        ],
    )
    return f(sidx, ones_h, zrows)


def _oddeven_comparators(n):
    res = []

    def merge(lo, m, r):
        step = r * 2
        if step < m:
            merge(lo, m, step)
            merge(lo + r, m, step)
            for i in range(lo + r, lo + m - r, step):
                res.append((i, i + r))
        else:
            res.append((lo, lo + r))

    def sort(lo, m):
        if m > 1:
            h = m // 2
            sort(lo, h)
            sort(lo + h, h)
            merge(lo, m, 1)

    sort(0, n)
    return res


_NSEG = 16


def _topk_kernel(hs_ref, ht_ref, o_ref, strip_ref, *, bn, n_valid):
    j = pl.program_id(1)
    nj = pl.num_programs(1)

    scores = jax.lax.dot_general(
        hs_ref[...], ht_ref[...], (((1,), (1,)), ((), ())),
        preferred_element_type=jnp.float32)

    col = jax.lax.broadcasted_iota(jnp.int32, scores.shape, 1) + j * bn
    scores = jnp.where(col < n_valid, scores, NEG)
    strip_ref[j] = scores

    @pl.when(j == nj - 1)
    def _select():
        a = [strip_ref[t] for t in range(_NSEG)]
        for (p, q) in _oddeven_comparators(_NSEG):
            hi = jnp.maximum(a[p], a[q])
            lo = jnp.minimum(a[p], a[q])
            a[p], a[q] = hi, lo
        w = a[0].shape[1]
        lane = jax.lax.broadcasted_iota(jnp.int32, a[0].shape, 1)
        vals = []
        for _ in range(K_TOP):
            m = jnp.max(a[0], axis=1, keepdims=True)
            vals.append(m)
            sel = jnp.where(a[0] == m, lane, w)
            first = jnp.min(sel, axis=1, keepdims=True)
            hit = lane == first
            for t in range(_NSEG - 1):
                a[t] = jnp.where(hit, a[t + 1], a[t])
            a[_NSEG - 1] = jnp.where(hit, NEG, a[_NSEG - 1])
        t = jnp.concatenate(vals, axis=1)
        e = jnp.exp(t - t[:, :1])
        s = jnp.sum(e, axis=1, keepdims=True)
        o_ref[...] = e / s


def _topk_softmax(h_s, h_t, bm=256, bn=640):
    m = h_s.shape[0]
    n = h_t.shape[0]
    mp = _round_up(m, bm)
    np_ = _round_up(n, bn * _NSEG) if n % (bn * _NSEG) else n
    if mp != m:
        h_s = jnp.pad(h_s, ((0, mp - m), (0, 0)))
    if np_ != n:
        h_t = jnp.pad(h_t, ((0, np_ - n), (0, 0)))
    assert np_ == bn * _NSEG
    out = pl.pallas_call(
        functools.partial(_topk_kernel, bn=bn, n_valid=n),
        grid=(mp // bm, _NSEG),
        in_specs=[
            pl.BlockSpec((bm, h_s.shape[1]), lambda i, j: (i, 0)),
            pl.BlockSpec((bn, h_t.shape[1]), lambda i, j: (j, 0)),
        ],
        out_specs=pl.BlockSpec((bm, K_TOP), lambda i, j: (i, 0)),
        out_shape=jax.ShapeDtypeStruct((mp, K_TOP), jnp.float32),
        scratch_shapes=[pltpu.VMEM((_NSEG, bm, bn), jnp.float32)],
        compiler_params=pltpu.CompilerParams(
            dimension_semantics=("parallel", "arbitrary")),
    )(h_s, h_t)
    return out[:m]


def _pad_idx(idx, fill):
    return jnp.concatenate([idx, fill.astype(jnp.int32)])


def kernel(x1, x2, edge1, edge2, y, params):
    e1 = jnp.concatenate([edge1[i] for i in range(edge1.shape[0])], axis=1)
    e2 = jnp.concatenate([edge2[i] for i in range(edge2.shape[0])], axis=1)
    n = x1.shape[0]
    x1p = jnp.pad(x1, ((0, NPAD - n), (0, 0)))
    x2p = jnp.pad(x2, ((0, NPAD - n), (0, 0)))
    x = jnp.concatenate([x1p, x2p], axis=0)

    npad_e = EPAD - E_EDGES
    ar = jnp.arange(npad_e, dtype=jnp.int32)
    gfill = ar % 64
    sfill = NPAD + (ar % NDUMP)
    src1, dst1 = e1[0], e1[1]
    src2, dst2 = e2[0], e2[1]
    gidx = jnp.stack([
        _pad_idx(src1, gfill),
        _pad_idx(dst1, gfill),
        _pad_idx(src2 + NPAD, gfill),
        _pad_idx(dst2 + NPAD, gfill),
    ]).reshape(4, EPAD // CHUNK, CHUNK)
    sidx = jnp.stack([
        _pad_idx(dst1, sfill),
        _pad_idx(src1, sfill),
        _pad_idx(dst2, sfill),
        _pad_idx(src2, sfill),
    ]).reshape(4, EPAD // CHUNK, CHUNK)

    zrows = jnp.zeros((NPAD + NDUMP, 128), jnp.float32)
    ones_h = jnp.ones((CHUNK, 128), jnp.float32)
    deg = _sc_degrees(sidx, ones_h, zrows)

    h = x
    xs = [x]
    for i in range(3):
        w12 = jnp.concatenate([params[f"W1_{i}"], params[f"W2_{i}"]], axis=1)
        mout = _mm4(h, w12)
        a1, a2 = _sc_scatter(mout, gidx, sidx, zrows)
        h = _combine(h, params[f"Wr_{i}"], params[f"br_{i}"], a1, a2, deg)
        xs.append(h)
    hcat = jnp.concatenate(xs, axis=1)
    hf = _mm(hcat, params["Wf"]) + params["bf"]
    h_s = hf[:n]
    h_t = hf[NPAD:NPAD + n]
    return _topk_softmax(h_s, h_t)
